# baseline (speedup 1.0000x reference)
import functools

import jax
import jax.numpy as jnp
from jax.experimental import pallas as pl
from jax.experimental.pallas import tpu as pltpu

_BF16 = jnp.bfloat16


def _gn_silu_bf16(xf, gamma, beta, num_groups, eps):
    S, C = xf.shape
    cpg = C // num_groups
    c_of = jax.lax.broadcasted_iota(jnp.int32, (C, num_groups), 0) // cpg
    g_of = jax.lax.broadcasted_iota(jnp.int32, (C, num_groups), 1)
    fwd = (c_of == g_of).astype(jnp.float32)
    g_oc = jax.lax.broadcasted_iota(jnp.int32, (num_groups, C), 0)
    c_oc = jax.lax.broadcasted_iota(jnp.int32, (num_groups, C), 1) // cpg
    bwd = (g_oc == c_oc).astype(jnp.float32)

    csum = jnp.sum(xf, axis=0, keepdims=True)
    csq = jnp.sum(xf * xf, axis=0, keepdims=True)
    both = jnp.concatenate([csum, csq], axis=0)
    gstats = jnp.dot(both, fwd,
                     preferred_element_type=jnp.float32) / jnp.float32(S * cpg)
    gmean = gstats[0:1]
    gvar = jnp.maximum(gstats[1:2] - gmean * gmean, 0.0)
    ginv = jax.lax.rsqrt(gvar + eps)
    cback = jnp.dot(jnp.concatenate([gmean, ginv], axis=0), bwd,
                    preferred_element_type=jnp.float32)
    scale = cback[1:2] * gamma
    shift = beta - cback[0:1] * scale
    y = xf * scale + shift
    return (y * jax.nn.sigmoid(y)).astype(_BF16)


def _store_padded(ref, y, T, H, W, C, KT, ph, pw):
    yv = y.reshape(T, H, W, C)
    ref[...] = jnp.zeros(ref.shape, ref.dtype)
    ref[KT - 1:KT - 1 + T, ph:ph + H, pw:pw + W, :] = yv
    if KT > 1:
        ref[0:KT - 1, ph:ph + H, pw:pw + W, :] = jnp.broadcast_to(
            yv[0:1], (KT - 1, H, W, C))


def _im2col(ref, T, H, W, KT, KH, KW):
    C = ref.shape[-1]
    S = T * H * W
    cols = []
    for kt in range(KT):
        for kh in range(KH):
            for kw in range(KW):
                cols.append(
                    ref[kt:kt + T, kh:kh + H, kw:kw + W, :].reshape(S, C))
    return jnp.concatenate(cols, axis=-1)


def _block_kernel(x_ref, g1_ref, b1_ref, w1_ref, cb1_ref, g2_ref, b2_ref,
                  w2_ref, cb2_ref, ninw_ref, o_ref, xp1_ref, xp2_ref, *,
                  num_groups, eps, T, H, W, K1, K2):
    KT1, KH1, KW1 = K1
    KT2, KH2, KW2 = K2
    xf = x_ref[0]

    y1 = _gn_silu_bf16(xf, g1_ref[...], b1_ref[...], num_groups, eps)
    _store_padded(xp1_ref, y1, T, H, W, y1.shape[-1], KT1, KH1 // 2, KW1 // 2)
    xc1 = _im2col(xp1_ref, T, H, W, KT1, KH1, KW1)
    h = jnp.dot(xc1, w1_ref[...], preferred_element_type=jnp.float32)
    h = h + cb1_ref[...]

    y2 = _gn_silu_bf16(h, g2_ref[...], b2_ref[...], num_groups, eps)
    _store_padded(xp2_ref, y2, T, H, W, y2.shape[-1], KT2, KH2 // 2, KW2 // 2)
    xc2 = _im2col(xp2_ref, T, H, W, KT2, KH2, KW2)
    acc = jnp.dot(xc2, w2_ref[...], preferred_element_type=jnp.float32)
    acc = acc + cb2_ref[...]
    acc = acc + jnp.dot(xf.astype(_BF16), ninw_ref[...],
                        preferred_element_type=jnp.float32)
    o_ref[0] = acc.astype(o_ref.dtype)


def kernel(x, norm1_gamma, norm1_beta, conv1_w, conv1_b, norm2_gamma,
           norm2_beta, conv2_w, conv2_b, nin_w, nin_b):
    N, Cin, T, H, W = x.shape
    S = T * H * W
    KT1, KH1, KW1, _, Cmid = conv1_w.shape
    KT2, KH2, KW2, _, Cout = conv2_w.shape
    num_groups, eps = 32, 1e-6

    xs = jnp.transpose(x, (0, 2, 3, 4, 1)).reshape(N, S, Cin)
    w1 = conv1_w.reshape(KT1 * KH1 * KW1 * Cin, Cmid).astype(_BF16)
    w2 = conv2_w.reshape(KT2 * KH2 * KW2 * Cmid, Cout).astype(_BF16)
    ninw = nin_w.astype(_BF16)
    cb2 = (conv2_b + nin_b).astype(jnp.float32).reshape(1, Cout)

    Tp1, Hp1, Wp1 = T + KT1 - 1, H + 2 * (KH1 // 2), W + 2 * (KW1 // 2)
    Tp2, Hp2, Wp2 = T + KT2 - 1, H + 2 * (KH2 // 2), W + 2 * (KW2 // 2)

    body = functools.partial(
        _block_kernel, num_groups=num_groups, eps=eps, T=T, H=H, W=W,
        K1=(KT1, KH1, KW1), K2=(KT2, KH2, KW2))

    out = pl.pallas_call(
        body,
        out_shape=jax.ShapeDtypeStruct((N, S, Cout), x.dtype),
        grid=(N,),
        in_specs=[
            pl.BlockSpec((1, S, Cin), lambda n: (n, 0, 0)),
            pl.BlockSpec((1, Cin), lambda n: (0, 0)),
            pl.BlockSpec((1, Cin), lambda n: (0, 0)),
            pl.BlockSpec((KT1 * KH1 * KW1 * Cin, Cmid), lambda n: (0, 0)),
            pl.BlockSpec((1, Cmid), lambda n: (0, 0)),
            pl.BlockSpec((1, Cmid), lambda n: (0, 0)),
            pl.BlockSpec((1, Cmid), lambda n: (0, 0)),
            pl.BlockSpec((KT2 * KH2 * KW2 * Cmid, Cout), lambda n: (0, 0)),
            pl.BlockSpec((1, Cout), lambda n: (0, 0)),
            pl.BlockSpec((Cin, Cout), lambda n: (0, 0)),
        ],
        out_specs=pl.BlockSpec((1, S, Cout), lambda n: (n, 0, 0)),
        scratch_shapes=[
            pltpu.VMEM((Tp1, Hp1, Wp1, Cin), _BF16),
            pltpu.VMEM((Tp2, Hp2, Wp2, Cmid), _BF16),
        ],
        compiler_params=pltpu.CompilerParams(
            dimension_semantics=("parallel",),
            vmem_limit_bytes=100 * 1024 * 1024,
        ),
    )(xs, norm1_gamma.reshape(1, Cin).astype(jnp.float32),
      norm1_beta.reshape(1, Cin).astype(jnp.float32), w1,
      conv1_b.astype(jnp.float32).reshape(1, Cmid),
      norm2_gamma.reshape(1, Cmid).astype(jnp.float32),
      norm2_beta.reshape(1, Cmid).astype(jnp.float32), w2, cb2, ninw)

    return jnp.transpose(out.reshape(N, T, H, W, Cout), (0, 4, 1, 2, 3))


# speedup vs baseline: 1.0522x; 1.0522x over previous
import functools

import jax
import jax.numpy as jnp
from jax.experimental import pallas as pl
from jax.experimental.pallas import tpu as pltpu

_BF16 = jnp.bfloat16


def _gn_silu_bf16(xf, gamma, beta, num_groups, eps, mask, count, mask_input):
    _, C = xf.shape
    cpg = C // num_groups
    c_of = jax.lax.broadcasted_iota(jnp.int32, (C, num_groups), 0) // cpg
    g_of = jax.lax.broadcasted_iota(jnp.int32, (C, num_groups), 1)
    fwd = (c_of == g_of).astype(jnp.float32)
    g_oc = jax.lax.broadcasted_iota(jnp.int32, (num_groups, C), 0)
    c_oc = jax.lax.broadcasted_iota(jnp.int32, (num_groups, C), 1) // cpg
    bwd = (g_oc == c_oc).astype(jnp.float32)

    xm = xf * mask if mask_input else xf
    csum = jnp.sum(xm, axis=0, keepdims=True)
    csq = jnp.sum(xf * xm, axis=0, keepdims=True)
    both = jnp.concatenate([csum, csq], axis=0)
    gstats = jnp.dot(both, fwd,
                     preferred_element_type=jnp.float32) / jnp.float32(
                         count * cpg)
    gmean = gstats[0:1]
    gvar = jnp.maximum(gstats[1:2] - gmean * gmean, 0.0)
    ginv = jax.lax.rsqrt(gvar + eps)
    cback = jnp.dot(jnp.concatenate([gmean, ginv], axis=0), bwd,
                    preferred_element_type=jnp.float32)
    scale = cback[1:2] * gamma
    shift = beta - cback[0:1] * scale
    y = xf * scale + shift
    return (y * jax.nn.sigmoid(y) * mask).astype(_BF16)


def _pad_conv_grid(xp_ref, ym, w_ref, KT, KH, KW, WP, FR, SR):
    OFF = (KT - 1) * FR + (KH // 2) * WP + (KW // 2)
    SHLEN = (KT - 1) * FR + (KH - 1) * WP + SR
    xp_ref[...] = jnp.zeros(xp_ref.shape, xp_ref.dtype)
    xp_ref[OFF:OFF + SR, :] = ym
    if KT > 1:
        rep = xp_ref[(KT - 1) * FR:KT * FR, :]
        for f in range(KT - 1):
            xp_ref[f * FR:(f + 1) * FR, :] = rep
    xv = xp_ref[...]
    xsh = [xv[k:k + SHLEN, :] for k in range(KW)]
    cols = []
    for kt in range(KT):
        for kh in range(KH):
            base = kt * FR + kh * WP
            for kw in range(KW):
                cols.append(xsh[kw][base:base + SR, :])
    xcat = jnp.concatenate(cols, axis=-1)
    return jnp.dot(xcat, w_ref[...], preferred_element_type=jnp.float32)


def _block_kernel(xg_ref, g1_ref, b1_ref, w1_ref, cb1_ref, g2_ref, b2_ref,
                  w2_ref, cb2_ref, ninw_ref, o_ref, xp1_ref, xp2_ref, *,
                  num_groups, eps, T, H, W, WP, KS):
    KT, KH, KW = KS
    HP = H + 2 * (KH // 2)
    FR = HP * WP
    SR = T * FR
    S = T * H * W
    Cout = o_ref.shape[-1]

    r = jax.lax.broadcasted_iota(jnp.int32, (SR, 1), 0)
    mask = ((r % WP < W) & (r % FR < H * WP)).astype(jnp.float32)

    xf = xg_ref[0]
    y1 = _gn_silu_bf16(xf, g1_ref[...], b1_ref[...], num_groups, eps,
                       mask, S, mask_input=False)
    h = _pad_conv_grid(xp1_ref, y1, w1_ref, KT, KH, KW, WP, FR, SR)
    h = h + cb1_ref[...]

    y2 = _gn_silu_bf16(h, g2_ref[...], b2_ref[...], num_groups, eps,
                       mask, S, mask_input=True)
    acc = _pad_conv_grid(xp2_ref, y2, w2_ref, KT, KH, KW, WP, FR, SR)
    acc = acc + cb2_ref[...]
    acc = acc + jnp.dot(xf.astype(_BF16), ninw_ref[...],
                        preferred_element_type=jnp.float32)

    o4 = acc.reshape(T, HP, WP, Cout)[:, :H, :W, :]
    o_ref[0] = o4.reshape(S, Cout).astype(o_ref.dtype)


def kernel(x, norm1_gamma, norm1_beta, conv1_w, conv1_b, norm2_gamma,
           norm2_beta, conv2_w, conv2_b, nin_w, nin_b):
    N, Cin, T, H, W = x.shape
    S = T * H * W
    KT, KH, KW, _, Cmid = conv1_w.shape
    Cout = conv2_w.shape[-1]
    num_groups, eps = 32, 1e-6

    HP = H + 2 * (KH // 2)
    WP = ((W + 2 * (KW // 2) + 7) // 8) * 8
    FR = HP * WP
    SR = T * FR
    SHLEN = (KT - 1) * FR + (KH - 1) * WP + SR
    RTOT = ((SHLEN + KW - 1 + 7) // 8) * 8

    xt = jnp.transpose(x, (0, 2, 3, 4, 1))
    xg = jnp.pad(xt, ((0, 0), (0, 0), (0, HP - H), (0, WP - W), (0, 0)))
    xg = xg.reshape(N, SR, Cin)

    w1 = conv1_w.reshape(KT * KH * KW * Cin, Cmid).astype(_BF16)
    w2 = conv2_w.reshape(KT * KH * KW * Cmid, Cout).astype(_BF16)
    ninw = nin_w.astype(_BF16)
    cb2 = (conv2_b + nin_b).astype(jnp.float32).reshape(1, Cout)

    body = functools.partial(
        _block_kernel, num_groups=num_groups, eps=eps, T=T, H=H, W=W,
        WP=WP, KS=(KT, KH, KW))

    out = pl.pallas_call(
        body,
        out_shape=jax.ShapeDtypeStruct((N, S, Cout), x.dtype),
        grid=(N,),
        in_specs=[
            pl.BlockSpec((1, SR, Cin), lambda n: (n, 0, 0)),
            pl.BlockSpec((1, Cin), lambda n: (0, 0)),
            pl.BlockSpec((1, Cin), lambda n: (0, 0)),
            pl.BlockSpec((KT * KH * KW * Cin, Cmid), lambda n: (0, 0)),
            pl.BlockSpec((1, Cmid), lambda n: (0, 0)),
            pl.BlockSpec((1, Cmid), lambda n: (0, 0)),
            pl.BlockSpec((1, Cmid), lambda n: (0, 0)),
            pl.BlockSpec((KT * KH * KW * Cmid, Cout), lambda n: (0, 0)),
            pl.BlockSpec((1, Cout), lambda n: (0, 0)),
            pl.BlockSpec((Cin, Cout), lambda n: (0, 0)),
        ],
        out_specs=pl.BlockSpec((1, S, Cout), lambda n: (n, 0, 0)),
        scratch_shapes=[
            pltpu.VMEM((RTOT, Cin), _BF16),
            pltpu.VMEM((RTOT, Cmid), _BF16),
        ],
        compiler_params=pltpu.CompilerParams(
            dimension_semantics=("parallel",),
            vmem_limit_bytes=100 * 1024 * 1024,
        ),
    )(xg, norm1_gamma.reshape(1, Cin).astype(jnp.float32),
      norm1_beta.reshape(1, Cin).astype(jnp.float32), w1,
      conv1_b.astype(jnp.float32).reshape(1, Cmid),
      norm2_gamma.reshape(1, Cmid).astype(jnp.float32),
      norm2_beta.reshape(1, Cmid).astype(jnp.float32), w2, cb2, ninw)

    return jnp.transpose(out.reshape(N, T, H, W, Cout), (0, 4, 1, 2, 3))


# speedup vs baseline: 1.0529x; 1.0006x over previous
import functools

import jax
import jax.numpy as jnp
from jax.experimental import pallas as pl
from jax.experimental.pallas import tpu as pltpu

_BF16 = jnp.bfloat16


def _gn_silu_bf16(xf, gamma, beta, num_groups, eps, mask, count, mask_input):
    _, C = xf.shape
    cpg = C // num_groups
    c_of = jax.lax.broadcasted_iota(jnp.int32, (C, num_groups), 0) // cpg
    g_of = jax.lax.broadcasted_iota(jnp.int32, (C, num_groups), 1)
    fwd = (c_of == g_of).astype(jnp.float32)
    g_oc = jax.lax.broadcasted_iota(jnp.int32, (num_groups, C), 0)
    c_oc = jax.lax.broadcasted_iota(jnp.int32, (num_groups, C), 1) // cpg
    bwd = (g_oc == c_oc).astype(jnp.float32)

    xm = xf * mask if mask_input else xf
    csum = jnp.sum(xm, axis=0, keepdims=True)
    csq = jnp.sum(xf * xm, axis=0, keepdims=True)
    both = jnp.concatenate([csum, csq], axis=0)
    gstats = jnp.dot(both, fwd,
                     preferred_element_type=jnp.float32) / jnp.float32(
                         count * cpg)
    gmean = gstats[0:1]
    gvar = jnp.maximum(gstats[1:2] - gmean * gmean, 0.0)
    ginv = jax.lax.rsqrt(gvar + eps)
    cback = jnp.dot(jnp.concatenate([gmean, ginv], axis=0), bwd,
                    preferred_element_type=jnp.float32)
    scale = cback[1:2] * gamma
    shift = beta - cback[0:1] * scale
    y = xf * scale + shift
    return (y * jax.nn.sigmoid(y) * mask).astype(_BF16)


def _pad_conv_grid(xp_ref, ym, w_ref, KT, KH, KW, WP, FR, SR):
    OFF = (KT - 1) * FR + (KH // 2) * WP + (KW // 2)
    SHLEN = (KT - 1) * FR + (KH - 1) * WP + SR
    xp_ref[...] = jnp.zeros(xp_ref.shape, xp_ref.dtype)
    xp_ref[OFF:OFF + SR, :] = ym
    if KT > 1:
        rep = xp_ref[(KT - 1) * FR:KT * FR, :]
        for f in range(KT - 1):
            xp_ref[f * FR:(f + 1) * FR, :] = rep
    xv = xp_ref[...]
    xsh = [xv[k:k + SHLEN, :] for k in range(KW)]
    cols = []
    for kt in range(KT):
        for kh in range(KH):
            base = kt * FR + kh * WP
            for kw in range(KW):
                cols.append(xsh[kw][base:base + SR, :])
    xcat = jnp.concatenate(cols, axis=-1)
    return jnp.dot(xcat, w_ref[...], preferred_element_type=jnp.float32)


def _block_kernel(xg_ref, g1_ref, b1_ref, w1_ref, cb1_ref, g2_ref, b2_ref,
                  w2_ref, cb2_ref, ninw_ref, o_ref, *scratch,
                  num_groups, eps, T, H, W, WP, KS, PER_STEP):
    KT, KH, KW = KS
    HP = H + 2 * (KH // 2)
    FR = HP * WP
    SR = T * FR
    S = T * H * W
    Cout = o_ref.shape[-1]

    r = jax.lax.broadcasted_iota(jnp.int32, (SR, 1), 0)
    mask = ((r % WP < W) & (r % FR < H * WP)).astype(jnp.float32)

    for i in range(PER_STEP):
        xp1_ref = scratch[i]
        xp2_ref = scratch[PER_STEP + i]
        xf = xg_ref[i]
        y1 = _gn_silu_bf16(xf, g1_ref[...], b1_ref[...], num_groups, eps,
                           mask, S, mask_input=False)
        h = _pad_conv_grid(xp1_ref, y1, w1_ref, KT, KH, KW, WP, FR, SR)
        h = h + cb1_ref[...]

        y2 = _gn_silu_bf16(h, g2_ref[...], b2_ref[...], num_groups, eps,
                           mask, S, mask_input=True)
        acc = _pad_conv_grid(xp2_ref, y2, w2_ref, KT, KH, KW, WP, FR, SR)
        acc = acc + cb2_ref[...]
        acc = acc + jnp.dot(xf.astype(_BF16), ninw_ref[...],
                            preferred_element_type=jnp.float32)

        o4 = acc.reshape(T, HP, WP, Cout)[:, :H, :W, :]
        o_ref[i] = o4.reshape(S, Cout).astype(o_ref.dtype)


def kernel(x, norm1_gamma, norm1_beta, conv1_w, conv1_b, norm2_gamma,
           norm2_beta, conv2_w, conv2_b, nin_w, nin_b):
    N, Cin, T, H, W = x.shape
    S = T * H * W
    KT, KH, KW, _, Cmid = conv1_w.shape
    Cout = conv2_w.shape[-1]
    num_groups, eps = 32, 1e-6

    HP = H + 2 * (KH // 2)
    WP = ((W + 2 * (KW // 2) + 7) // 8) * 8
    FR = HP * WP
    SR = T * FR
    SHLEN = (KT - 1) * FR + (KH - 1) * WP + SR
    RTOT = ((SHLEN + KW - 1 + 7) // 8) * 8

    xt = jnp.transpose(x, (0, 2, 3, 4, 1))
    xg = jnp.pad(xt, ((0, 0), (0, 0), (0, HP - H), (0, WP - W), (0, 0)))
    xg = xg.reshape(N, SR, Cin)

    w1 = conv1_w.reshape(KT * KH * KW * Cin, Cmid).astype(_BF16)
    w2 = conv2_w.reshape(KT * KH * KW * Cmid, Cout).astype(_BF16)
    ninw = nin_w.astype(_BF16)
    cb2 = (conv2_b + nin_b).astype(jnp.float32).reshape(1, Cout)

    PER_STEP = 2 if N % 2 == 0 else 1
    body = functools.partial(
        _block_kernel, num_groups=num_groups, eps=eps, T=T, H=H, W=W,
        WP=WP, KS=(KT, KH, KW), PER_STEP=PER_STEP)

    out = pl.pallas_call(
        body,
        out_shape=jax.ShapeDtypeStruct((N, S, Cout), x.dtype),
        grid=(N // PER_STEP,),
        in_specs=[
            pl.BlockSpec((PER_STEP, SR, Cin), lambda n: (n, 0, 0)),
            pl.BlockSpec((1, Cin), lambda n: (0, 0)),
            pl.BlockSpec((1, Cin), lambda n: (0, 0)),
            pl.BlockSpec((KT * KH * KW * Cin, Cmid), lambda n: (0, 0)),
            pl.BlockSpec((1, Cmid), lambda n: (0, 0)),
            pl.BlockSpec((1, Cmid), lambda n: (0, 0)),
            pl.BlockSpec((1, Cmid), lambda n: (0, 0)),
            pl.BlockSpec((KT * KH * KW * Cmid, Cout), lambda n: (0, 0)),
            pl.BlockSpec((1, Cout), lambda n: (0, 0)),
            pl.BlockSpec((Cin, Cout), lambda n: (0, 0)),
        ],
        out_specs=pl.BlockSpec((PER_STEP, S, Cout), lambda n: (n, 0, 0)),
        scratch_shapes=(
            [pltpu.VMEM((RTOT, Cin), _BF16) for _ in range(PER_STEP)]
            + [pltpu.VMEM((RTOT, Cmid), _BF16) for _ in range(PER_STEP)]),
        compiler_params=pltpu.CompilerParams(
            dimension_semantics=("parallel",),
            vmem_limit_bytes=100 * 1024 * 1024,
        ),
    )(xg, norm1_gamma.reshape(1, Cin).astype(jnp.float32),
      norm1_beta.reshape(1, Cin).astype(jnp.float32), w1,
      conv1_b.astype(jnp.float32).reshape(1, Cmid),
      norm2_gamma.reshape(1, Cmid).astype(jnp.float32),
      norm2_beta.reshape(1, Cmid).astype(jnp.float32), w2, cb2, ninw)

    return jnp.transpose(out.reshape(N, T, H, W, Cout), (0, 4, 1, 2, 3))


# speedup vs baseline: 1.3587x; 1.2905x over previous
import functools

import jax
import jax.numpy as jnp
from jax.experimental import pallas as pl
from jax.experimental.pallas import tpu as pltpu

_BF16 = jnp.bfloat16


def _gn_silu_bf16(xf, gamma, beta, num_groups, eps, mask, count, mask_input):
    _, C = xf.shape
    cpg = C // num_groups
    denom = jnp.float32(count * cpg)

    xm = xf * mask if mask_input else xf
    csum = jnp.sum(xm, axis=0, keepdims=True)
    csq = jnp.sum(xf * xm, axis=0, keepdims=True)
    lane = jax.lax.broadcasted_iota(jnp.int32, (1, C), 1)

    def _seg_allsum(v):
        s = 1
        while s < cpg:
            partner = jnp.where((lane & s) == 0,
                                jnp.roll(v, -s, axis=1),
                                jnp.roll(v, s, axis=1))
            v = v + partner
            s *= 2
        return v

    mean_c = _seg_allsum(csum) / denom
    ex2_c = _seg_allsum(csq) / denom
    var_c = jnp.maximum(ex2_c - mean_c * mean_c, 0.0)
    inv_c = jax.lax.rsqrt(var_c + eps)
    scale = inv_c * gamma
    shift = beta - mean_c * scale
    y = xf * scale + shift
    return (y * jax.nn.sigmoid(y) * mask).astype(_BF16)


def _store_shifted(xp_ref, ym, C, NSH, KT, OFF, FR, SR):
    xp_ref[...] = jnp.zeros(xp_ref.shape, xp_ref.dtype)
    for k in range(NSH):
        xp_ref[OFF - k:OFF - k + SR, k * C:(k + 1) * C] = ym
    if KT > 1:
        rep = xp_ref[(KT - 1) * FR:KT * FR, :]
        for f in range(KT - 1):
            xp_ref[f * FR:(f + 1) * FR, :] = rep


def _mrb_conv(pairs, M):
    per_mxu = [0, 0]
    for i, (lhs, rhs) in enumerate(pairs):
        mx = i % 2
        sr = per_mxu[mx] % 2
        pltpu.matmul_push_rhs(rhs, staging_register=sr, mxu_index=mx)
        pltpu.matmul_acc_lhs(acc_addr=0, lhs=lhs, mxu_index=mx,
                             load_staged_rhs=sr)
        per_mxu[mx] += 1
    r0 = pltpu.matmul_pop(acc_addr=0, shape=(M, 256), dtype=jnp.float32,
                          mxu_index=0)
    r1 = pltpu.matmul_pop(acc_addr=0, shape=(M, 256), dtype=jnp.float32,
                          mxu_index=1)
    return r0 + r1


def _conv_pairs(xp_ref, w_ref, KT, KH, FR, WP, SR):
    n_lblk = xp_ref.shape[-1] // 256
    pairs = []
    t_idx = 0
    for kt in range(KT):
        for kh in range(KH):
            base = kt * FR + kh * WP
            for b in range(n_lblk):
                pairs.append(
                    (xp_ref[base:base + SR, b * 256:(b + 1) * 256],
                     w_ref[t_idx * 256:(t_idx + 1) * 256, :]))
                t_idx += 1
    return pairs


def _block_kernel(xg_ref, g1_ref, b1_ref, w1_ref, cb1_ref, g2_ref, b2_ref,
                  w2_ref, cb2_ref, ninw_ref, o_ref, xp1_ref, xp2_ref, *,
                  num_groups, eps, T, H, W, WP, KS, Cin, Cmid, Cout):
    KT, KH, KW = KS
    HP = H + 2 * (KH // 2)
    FR = HP * WP
    SR = T * FR
    S = T * H * W
    OFF = (KT - 1) * FR + (KH // 2) * WP + (KW // 2)

    r = jax.lax.broadcasted_iota(jnp.int32, (SR, 1), 0)
    mask = ((r % WP < W) & (r % FR < H * WP)).astype(jnp.float32)

    xf = xg_ref[0]

    y1 = _gn_silu_bf16(xf, g1_ref[...], b1_ref[...], num_groups, eps,
                       mask, S, mask_input=False)
    _store_shifted(xp1_ref, y1, Cin, KW, KT, OFF, FR, SR)
    h = _mrb_conv(_conv_pairs(xp1_ref, w1_ref, KT, KH, FR, WP, SR), SR)
    h = h + cb1_ref[...]

    y2 = _gn_silu_bf16(h, g2_ref[...], b2_ref[...], num_groups, eps,
                       mask, S, mask_input=True)
    _store_shifted(xp2_ref, y2, Cmid, KW, KT, OFF, FR, SR)
    pairs = _conv_pairs(xp2_ref, w2_ref, KT, KH, FR, WP, SR)
    xnin = jnp.concatenate(
        [xf.astype(_BF16),
         jnp.zeros((SR, 256 - Cin), _BF16)], axis=-1) if Cin < 256 else \
        xf.astype(_BF16)
    pairs.append((xnin, ninw_ref[...]))
    acc = _mrb_conv(pairs, SR)
    acc = acc + cb2_ref[...]

    o4 = acc.reshape(T, HP, WP, Cout)[:, :H, :W, :]
    o_ref[0] = o4.reshape(S, Cout).astype(o_ref.dtype)


def kernel(x, norm1_gamma, norm1_beta, conv1_w, conv1_b, norm2_gamma,
           norm2_beta, conv2_w, conv2_b, nin_w, nin_b):
    N, Cin, T, H, W = x.shape
    S = T * H * W
    KT, KH, KW, _, Cmid = conv1_w.shape
    Cout = conv2_w.shape[-1]
    num_groups, eps = 32, 1e-6

    HP = H + 2 * (KH // 2)
    WP = ((W + 2 * (KW // 2) + 7) // 8) * 8
    FR = HP * WP
    SR = T * FR
    SHLEN = (KT - 1) * FR + (KH - 1) * WP + SR
    RTOT = ((SHLEN + KW - 1 + 7) // 8) * 8

    lw1 = ((KW * Cin + 255) // 256) * 256
    lw2 = ((KW * Cmid + 255) // 256) * 256

    xt = jnp.transpose(x, (0, 2, 3, 4, 1))
    xg = jnp.pad(xt, ((0, 0), (0, 0), (0, HP - H), (0, WP - W), (0, 0)))
    xg = xg.reshape(N, SR, Cin)

    def _tile_weights(w, lw):
        kT, kH, kW, c, co = w.shape
        wg = w.astype(_BF16).reshape(kT * kH, kW * c, co)
        wg = jnp.pad(wg, ((0, 0), (0, lw - kW * c), (0, 0)))
        return wg.reshape(-1, co)

    w1e = _tile_weights(conv1_w, lw1)
    w2e = _tile_weights(conv2_w, lw2)
    nine = jnp.concatenate(
        [nin_w.astype(_BF16),
         jnp.zeros((256 - Cin, Cout), _BF16)], axis=0) if Cin < 256 else \
        nin_w.astype(_BF16)
    cb2 = (conv2_b + nin_b).astype(jnp.float32).reshape(1, Cout)

    body = functools.partial(
        _block_kernel, num_groups=num_groups, eps=eps, T=T, H=H, W=W,
        WP=WP, KS=(KT, KH, KW), Cin=Cin, Cmid=Cmid, Cout=Cout)

    out = pl.pallas_call(
        body,
        out_shape=jax.ShapeDtypeStruct((N, S, Cout), x.dtype),
        grid=(N,),
        in_specs=[
            pl.BlockSpec((1, SR, Cin), lambda n: (n, 0, 0)),
            pl.BlockSpec((1, Cin), lambda n: (0, 0)),
            pl.BlockSpec((1, Cin), lambda n: (0, 0)),
            pl.BlockSpec(w1e.shape, lambda n: (0, 0)),
            pl.BlockSpec((1, Cmid), lambda n: (0, 0)),
            pl.BlockSpec((1, Cmid), lambda n: (0, 0)),
            pl.BlockSpec((1, Cmid), lambda n: (0, 0)),
            pl.BlockSpec(w2e.shape, lambda n: (0, 0)),
            pl.BlockSpec((1, Cout), lambda n: (0, 0)),
            pl.BlockSpec((256, Cout), lambda n: (0, 0)),
        ],
        out_specs=pl.BlockSpec((1, S, Cout), lambda n: (n, 0, 0)),
        scratch_shapes=[
            pltpu.VMEM((RTOT, lw1), _BF16),
            pltpu.VMEM((RTOT, lw2), _BF16),
        ],
        compiler_params=pltpu.CompilerParams(
            dimension_semantics=("parallel",),
            vmem_limit_bytes=100 * 1024 * 1024,
        ),
    )(xg, norm1_gamma.reshape(1, Cin).astype(jnp.float32),
      norm1_beta.reshape(1, Cin).astype(jnp.float32), w1e,
      conv1_b.astype(jnp.float32).reshape(1, Cmid),
      norm2_gamma.reshape(1, Cmid).astype(jnp.float32),
      norm2_beta.reshape(1, Cmid).astype(jnp.float32), w2e, cb2, nine)

    return jnp.transpose(out.reshape(N, T, H, W, Cout), (0, 4, 1, 2, 3))
